# baseline (speedup 1.0000x reference)
import functools

import jax
import jax.numpy as jnp
from jax import lax
from jax.experimental import pallas as pl
from jax.experimental.pallas import tpu as pltpu
from jax.experimental.pallas import tpu_sc as plsc

N = 10000
E = 320000
D = 128
NC = 2
NS = 16
NW = NC * NS
CHUNK = 96
NCH = 105
EPW = NCH * CHUNK
E_PAD = NW * EPW
NDUMMY = 16
NPAD = 10240
RPT = NPAD // NS
LAST_RPT = N - 15 * RPT
ZR = 64
BR = 1000


def _sc_mesh():
    return plsc.VectorSubcoreMesh(core_axis_name="c", subcore_axis_name="s")


def _sc_degree(dst3d, ones128, zeros128):
    @functools.partial(
        pl.kernel,
        out_type=jax.ShapeDtypeStruct((NC, N, D), jnp.float32),
        mesh=_sc_mesh(),
        scratch_types=[
            pltpu.VMEM((NCH, CHUNK), jnp.int32),
            pltpu.VMEM((CHUNK, D), jnp.float32),
            pltpu.VMEM_SHARED((NPAD, D), jnp.float32),
            pltpu.SemaphoreType.DMA,
            pltpu.SemaphoreType.DMA,
        ],
    )
    def k(dst_hbm, ones_hbm, z_hbm, out_hbm, dst_v, ones_v, acc, sem0, sem1):
        cid = lax.axis_index("c")
        sid = lax.axis_index("s")
        wid = sid * NC + cid
        r0 = sid * RPT
        pltpu.sync_copy(z_hbm, ones_v.at[pl.ds(0, ZR)])
        for j in range(RPT // ZR):
            pltpu.sync_copy(ones_v.at[pl.ds(0, ZR)],
                            acc.at[pl.ds(r0 + j * ZR, ZR)])
        pltpu.sync_copy(ones_hbm, ones_v)
        pltpu.sync_copy(dst_hbm.at[wid], dst_v)
        plsc.subcore_barrier()

        sems = (sem0, sem1)

        def scat(i, b):
            pltpu.async_copy(ones_v, acc.at[dst_v.at[i]], sems[b], add=True)

        def swait(b):
            pltpu.make_async_copy(ones_hbm, ones_v, sems[b]).wait()

        scat(0, 0)
        scat(1, 1)

        @pl.loop(2, NCH - 1, step=2)
        def _(g):
            for b in range(2):
                i = g + b
                swait(b)
                scat(i, b)

        i = NCH - 1
        swait(0)
        scat(i, 0)
        swait(1)
        swait(0)
        plsc.subcore_barrier()

        @pl.when(sid < NS - 1)
        def _():
            pltpu.sync_copy(acc.at[pl.ds(r0, RPT)],
                            out_hbm.at[cid, pl.ds(r0, RPT)])

        @pl.when(sid == NS - 1)
        def _():
            pltpu.sync_copy(acc.at[pl.ds(r0, LAST_RPT)],
                            out_hbm.at[cid, pl.ds(r0, LAST_RPT)])

    return k(dst3d, ones128, zeros128)


def _sc_spmm(xs, src2d, dst3d, zeros128):
    @functools.partial(
        pl.kernel,
        out_type=jax.ShapeDtypeStruct((NC, N, D), jnp.float32),
        mesh=_sc_mesh(),
        scratch_types=[
            pltpu.VMEM((EPW,), jnp.int32),
            pltpu.VMEM((NCH, CHUNK), jnp.int32),
            pltpu.VMEM((CHUNK, D), jnp.float32),
            pltpu.VMEM((CHUNK, D), jnp.float32),
            pltpu.VMEM_SHARED((NPAD, D), jnp.float32),
            pltpu.SemaphoreType.DMA,
            pltpu.SemaphoreType.DMA,
            pltpu.SemaphoreType.DMA,
            pltpu.SemaphoreType.DMA,
        ],
    )
    def k(xs_hbm, src_hbm, dst_hbm, z_hbm, out_hbm,
          src_v, dst_v, rows0_v, rows1_v, acc, gsem0, gsem1, ssem0, ssem1):
        cid = lax.axis_index("c")
        sid = lax.axis_index("s")
        wid = sid * NC + cid
        r0 = sid * RPT
        pltpu.sync_copy(z_hbm, rows0_v.at[pl.ds(0, ZR)])
        for j in range(RPT // ZR):
            pltpu.sync_copy(rows0_v.at[pl.ds(0, ZR)],
                            acc.at[pl.ds(r0 + j * ZR, ZR)])
        pltpu.sync_copy(src_hbm.at[wid], src_v)
        pltpu.sync_copy(dst_hbm.at[wid], dst_v)
        plsc.subcore_barrier()

        bufs = (rows0_v, rows1_v)
        gsems = (gsem0, gsem1)
        ssems = (ssem0, ssem1)

        def gather(i, b):
            pltpu.async_copy(
                xs_hbm.at[src_v.at[pl.ds(i * CHUNK, CHUNK)]], bufs[b], gsems[b]
            )

        def gwait(b):
            pltpu.make_async_copy(xs_hbm.at[pl.ds(0, CHUNK)], bufs[b],
                                  gsems[b]).wait()

        def scat(i, b):
            pltpu.async_copy(bufs[b], acc.at[dst_v.at[i]], ssems[b], add=True)

        def swait(b):
            pltpu.make_async_copy(xs_hbm.at[pl.ds(0, CHUNK)], bufs[b],
                                  ssems[b]).wait()

        gather(0, 0)
        gwait(0)
        scat(0, 0)
        gather(1, 1)

        @pl.loop(1, NCH - 2, step=2)
        def _(g):
            for b in range(2):
                i = g + b
                buf = (1 + b) % 2
                gwait(buf)
                scat(i, buf)
                swait(1 - buf)
                gather(i + 1, 1 - buf)

        i = NCH - 2
        gwait(i % 2)
        scat(i, i % 2)
        swait((i + 1) % 2)
        gather(i + 1, (i + 1) % 2)
        i = NCH - 1
        gwait(i % 2)
        scat(i, i % 2)
        swait((i + 1) % 2)
        swait(i % 2)
        plsc.subcore_barrier()

        @pl.when(sid < NS - 1)
        def _():
            pltpu.sync_copy(acc.at[pl.ds(r0, RPT)],
                            out_hbm.at[cid, pl.ds(r0, RPT)])

        @pl.when(sid == NS - 1)
        def _():
            pltpu.sync_copy(acc.at[pl.ds(r0, LAST_RPT)],
                            out_hbm.at[cid, pl.ds(r0, LAST_RPT)])

    return k(xs, src2d, dst3d, zeros128)


def _tc_norm_scale(deg2, h):
    def body(deg_ref, h_ref, xs_ref, nb_ref):
        d = deg_ref[0, :, 0:1] + deg_ref[1, :, 0:1]
        norm = jnp.where(d > 0.0, lax.rsqrt(jnp.maximum(d, 1.0)), 0.0)
        nb = jnp.broadcast_to(norm, (BR, D))
        nb_ref[...] = nb
        xs_ref[...] = h_ref[...] * nb

    return pl.pallas_call(
        body,
        grid=(N // BR,),
        in_specs=[
            pl.BlockSpec((NC, BR, D), lambda i: (0, i, 0)),
            pl.BlockSpec((BR, D), lambda i: (i, 0)),
        ],
        out_specs=[
            pl.BlockSpec((BR, D), lambda i: (i, 0)),
            pl.BlockSpec((BR, D), lambda i: (i, 0)),
        ],
        out_shape=[
            jax.ShapeDtypeStruct((N, D), jnp.float32),
            jax.ShapeDtypeStruct((N, D), jnp.float32),
        ],
    )(deg2, h)


def _tc_layer(agg2, nb, W, b):
    def body(a_ref, nb_ref, w_ref, b_ref, o_ref):
        nbv = nb_ref[...]
        a = (a_ref[0] + a_ref[1]) * nbv
        z = jnp.dot(a, w_ref[...], preferred_element_type=jnp.float32)
        o_ref[...] = jnp.maximum(z + b_ref[...], 0.0) * nbv

    return pl.pallas_call(
        body,
        grid=(N // BR,),
        in_specs=[
            pl.BlockSpec((NC, BR, D), lambda i: (0, i, 0)),
            pl.BlockSpec((BR, D), lambda i: (i, 0)),
            pl.BlockSpec((D, D), lambda i: (0, 0)),
            pl.BlockSpec((1, D), lambda i: (0, 0)),
        ],
        out_specs=pl.BlockSpec((BR, D), lambda i: (i, 0)),
        out_shape=jax.ShapeDtypeStruct((N, D), jnp.float32),
    )(agg2, nb, W, b)


def _tc_final(agg2, nb, W, b, Wcp, bcp):
    def body(a_ref, nb_ref, w_ref, b_ref, wc_ref, bc_ref, pool_ref, o_ref):
        i = pl.program_id(0)
        a = (a_ref[0] + a_ref[1]) * nb_ref[...]
        z = jnp.dot(a, w_ref[...], preferred_element_type=jnp.float32)
        y = jnp.maximum(z + b_ref[...], 0.0)
        m = jnp.broadcast_to(jnp.max(y, axis=0, keepdims=True), (8, D))

        @pl.when(i == 0)
        def _():
            pool_ref[...] = m

        @pl.when(i > 0)
        def _():
            pool_ref[...] = jnp.maximum(pool_ref[...], m)

        @pl.when(i == pl.num_programs(0) - 1)
        def _():
            pooled = pool_ref[0:1, :]
            logits = jnp.dot(pooled, wc_ref[...],
                             preferred_element_type=jnp.float32) + bc_ref[...]
            o_ref[...] = jnp.broadcast_to(logits, (8, D))

    return pl.pallas_call(
        body,
        grid=(N // BR,),
        in_specs=[
            pl.BlockSpec((NC, BR, D), lambda i: (0, i, 0)),
            pl.BlockSpec((BR, D), lambda i: (i, 0)),
            pl.BlockSpec((D, D), lambda i: (0, 0)),
            pl.BlockSpec((1, D), lambda i: (0, 0)),
            pl.BlockSpec((D, D), lambda i: (0, 0)),
            pl.BlockSpec((1, D), lambda i: (0, 0)),
        ],
        out_specs=[
            pl.BlockSpec((8, D), lambda i: (0, 0)),
            pl.BlockSpec((8, D), lambda i: (0, 0)),
        ],
        out_shape=[
            jax.ShapeDtypeStruct((8, D), jnp.float32),
            jax.ShapeDtypeStruct((8, D), jnp.float32),
        ],
    )(agg2, nb, W, b, Wcp, bcp)


def kernel(h, edge_index, W1, b1, W2, b2, Wc, bc):
    src = edge_index[0]
    dst = edge_index[1]
    pad = E_PAD - E
    lane = jnp.arange(pad, dtype=jnp.int32) % NDUMMY
    src_p = jnp.concatenate([src, lane])
    dst_p = jnp.concatenate([dst, N + lane])
    src2d = src_p.reshape(NW, EPW)
    dst3d = dst_p.reshape(NW, NCH, CHUNK)

    ones128 = jnp.ones((CHUNK, D), jnp.float32)
    zeros128 = jnp.zeros((ZR, D), jnp.float32)

    deg2 = _sc_degree(dst3d, ones128, zeros128)
    xs, nb = _tc_norm_scale(deg2, h)
    agg1 = _sc_spmm(xs, src2d, dst3d, zeros128)
    y1s = _tc_layer(agg1, nb, W1, b1.reshape(1, D))
    agg2 = _sc_spmm(y1s, src2d, dst3d, zeros128)

    wcp = jnp.zeros((D, D), jnp.float32).at[:, : Wc.shape[1]].set(Wc)
    bcp = jnp.zeros((1, D), jnp.float32).at[0, : bc.shape[0]].set(bc)
    _, logits = _tc_final(agg2, nb, W2, b2.reshape(1, D), wcp, bcp)
    return logits[0:1, : Wc.shape[1]]


# speedup vs baseline: 1.1649x; 1.1649x over previous
import functools

import jax
import jax.numpy as jnp
from jax import lax
from jax.experimental import pallas as pl
from jax.experimental.pallas import tpu as pltpu
from jax.experimental.pallas import tpu_sc as plsc

N = 10000
E = 320000
D = 128
NC = 2
NS = 16
NW = NC * NS
CHUNK = 96
NCH = 105
EPW = NCH * CHUNK
E_PAD = NW * EPW
NDUMMY = 16
NPAD = 10240
RPT = NPAD // NS
LAST_RPT = N - 15 * RPT
ZR = 64
BR = 1000


def _sc_mesh():
    return plsc.VectorSubcoreMesh(core_axis_name="c", subcore_axis_name="s")


def _sc_degree(dst3d, ones128, zeros128):
    @functools.partial(
        pl.kernel,
        out_type=jax.ShapeDtypeStruct((NC, N, D), jnp.float32),
        mesh=_sc_mesh(),
        scratch_types=[
            pltpu.VMEM((NCH, CHUNK), jnp.int32),
            pltpu.VMEM((CHUNK, D), jnp.float32),
            pltpu.VMEM_SHARED((NPAD, D), jnp.float32),
        ],
    )
    def k(dst_hbm, ones_hbm, z_hbm, out_hbm, dst_v, ones_v, acc):
        cid = lax.axis_index("c")
        sid = lax.axis_index("s")
        wid = sid * NC + cid
        r0 = sid * RPT
        pltpu.sync_copy(z_hbm, ones_v.at[pl.ds(0, ZR)])
        for j in range(RPT // ZR):
            pltpu.sync_copy(ones_v.at[pl.ds(0, ZR)],
                            acc.at[pl.ds(r0 + j * ZR, ZR)])
        pltpu.sync_copy(ones_hbm, ones_v)
        pltpu.sync_copy(dst_hbm.at[wid], dst_v)
        plsc.subcore_barrier()

        def body(i, carry):
            pltpu.sync_copy(ones_v, acc.at[dst_v.at[i]], add=True)
            return carry

        lax.fori_loop(0, NCH, body, 0)
        plsc.subcore_barrier()

        @pl.when(sid < NS - 1)
        def _():
            pltpu.sync_copy(acc.at[pl.ds(r0, RPT)],
                            out_hbm.at[cid, pl.ds(r0, RPT)])

        @pl.when(sid == NS - 1)
        def _():
            pltpu.sync_copy(acc.at[pl.ds(r0, LAST_RPT)],
                            out_hbm.at[cid, pl.ds(r0, LAST_RPT)])

    return k(dst3d, ones128, zeros128)


def _sc_spmm(xs, src2d, dst3d, zeros128):
    @functools.partial(
        pl.kernel,
        out_type=jax.ShapeDtypeStruct((NC, N, D), jnp.float32),
        mesh=_sc_mesh(),
        scratch_types=[
            pltpu.VMEM((EPW,), jnp.int32),
            pltpu.VMEM((NCH, CHUNK), jnp.int32),
            pltpu.VMEM((CHUNK, D), jnp.float32),
            pltpu.VMEM((CHUNK, D), jnp.float32),
            pltpu.VMEM_SHARED((NPAD, D), jnp.float32),
            pltpu.SemaphoreType.DMA,
            pltpu.SemaphoreType.DMA,
        ],
    )
    def k(xs_hbm, src_hbm, dst_hbm, z_hbm, out_hbm,
          src_v, dst_v, rows0_v, rows1_v, acc, gsem0, gsem1):
        cid = lax.axis_index("c")
        sid = lax.axis_index("s")
        wid = sid * NC + cid
        r0 = sid * RPT
        pltpu.sync_copy(z_hbm, rows0_v.at[pl.ds(0, ZR)])
        for j in range(RPT // ZR):
            pltpu.sync_copy(rows0_v.at[pl.ds(0, ZR)],
                            acc.at[pl.ds(r0 + j * ZR, ZR)])
        pltpu.sync_copy(src_hbm.at[wid], src_v)
        pltpu.sync_copy(dst_hbm.at[wid], dst_v)
        plsc.subcore_barrier()

        bufs = (rows0_v, rows1_v)
        gsems = (gsem0, gsem1)

        def gather(i, b):
            pltpu.async_copy(
                xs_hbm.at[src_v.at[pl.ds(i * CHUNK, CHUNK)]], bufs[b], gsems[b]
            )

        def gwait(b):
            pltpu.make_async_copy(xs_hbm.at[pl.ds(0, CHUNK)], bufs[b],
                                  gsems[b]).wait()

        gather(0, 0)

        @pl.loop(0, NCH - 1, step=2)
        def _(g):
            for b in range(2):
                i = g + b
                gather(i + 1, 1 - b)
                gwait(b)
                pltpu.sync_copy(bufs[b], acc.at[dst_v.at[i]], add=True)

        gwait((NCH - 1) % 2)
        pltpu.sync_copy(bufs[(NCH - 1) % 2],
                        acc.at[dst_v.at[NCH - 1]], add=True)
        plsc.subcore_barrier()

        @pl.when(sid < NS - 1)
        def _():
            pltpu.sync_copy(acc.at[pl.ds(r0, RPT)],
                            out_hbm.at[cid, pl.ds(r0, RPT)])

        @pl.when(sid == NS - 1)
        def _():
            pltpu.sync_copy(acc.at[pl.ds(r0, LAST_RPT)],
                            out_hbm.at[cid, pl.ds(r0, LAST_RPT)])

    return k(xs, src2d, dst3d, zeros128)


def _tc_norm_scale(deg2, h):
    def body(deg_ref, h_ref, xs_ref, nb_ref):
        d = deg_ref[0, :, 0:1] + deg_ref[1, :, 0:1]
        norm = jnp.where(d > 0.0, lax.rsqrt(jnp.maximum(d, 1.0)), 0.0)
        nb = jnp.broadcast_to(norm, (BR, D))
        nb_ref[...] = nb
        xs_ref[...] = h_ref[...] * nb

    return pl.pallas_call(
        body,
        grid=(N // BR,),
        in_specs=[
            pl.BlockSpec((NC, BR, D), lambda i: (0, i, 0)),
            pl.BlockSpec((BR, D), lambda i: (i, 0)),
        ],
        out_specs=[
            pl.BlockSpec((BR, D), lambda i: (i, 0)),
            pl.BlockSpec((BR, D), lambda i: (i, 0)),
        ],
        out_shape=[
            jax.ShapeDtypeStruct((N, D), jnp.float32),
            jax.ShapeDtypeStruct((N, D), jnp.float32),
        ],
    )(deg2, h)


def _tc_layer(agg2, nb, W, b):
    def body(a_ref, nb_ref, w_ref, b_ref, o_ref):
        nbv = nb_ref[...]
        a = (a_ref[0] + a_ref[1]) * nbv
        z = jnp.dot(a, w_ref[...], preferred_element_type=jnp.float32)
        o_ref[...] = jnp.maximum(z + b_ref[...], 0.0) * nbv

    return pl.pallas_call(
        body,
        grid=(N // BR,),
        in_specs=[
            pl.BlockSpec((NC, BR, D), lambda i: (0, i, 0)),
            pl.BlockSpec((BR, D), lambda i: (i, 0)),
            pl.BlockSpec((D, D), lambda i: (0, 0)),
            pl.BlockSpec((1, D), lambda i: (0, 0)),
        ],
        out_specs=pl.BlockSpec((BR, D), lambda i: (i, 0)),
        out_shape=jax.ShapeDtypeStruct((N, D), jnp.float32),
    )(agg2, nb, W, b)


def _tc_final(agg2, nb, W, b, Wcp, bcp):
    def body(a_ref, nb_ref, w_ref, b_ref, wc_ref, bc_ref, pool_ref, o_ref):
        i = pl.program_id(0)
        a = (a_ref[0] + a_ref[1]) * nb_ref[...]
        z = jnp.dot(a, w_ref[...], preferred_element_type=jnp.float32)
        y = jnp.maximum(z + b_ref[...], 0.0)
        m = jnp.broadcast_to(jnp.max(y, axis=0, keepdims=True), (8, D))

        @pl.when(i == 0)
        def _():
            pool_ref[...] = m

        @pl.when(i > 0)
        def _():
            pool_ref[...] = jnp.maximum(pool_ref[...], m)

        @pl.when(i == pl.num_programs(0) - 1)
        def _():
            pooled = pool_ref[0:1, :]
            logits = jnp.dot(pooled, wc_ref[...],
                             preferred_element_type=jnp.float32) + bc_ref[...]
            o_ref[...] = jnp.broadcast_to(logits, (8, D))

    return pl.pallas_call(
        body,
        grid=(N // BR,),
        in_specs=[
            pl.BlockSpec((NC, BR, D), lambda i: (0, i, 0)),
            pl.BlockSpec((BR, D), lambda i: (i, 0)),
            pl.BlockSpec((D, D), lambda i: (0, 0)),
            pl.BlockSpec((1, D), lambda i: (0, 0)),
            pl.BlockSpec((D, D), lambda i: (0, 0)),
            pl.BlockSpec((1, D), lambda i: (0, 0)),
        ],
        out_specs=[
            pl.BlockSpec((8, D), lambda i: (0, 0)),
            pl.BlockSpec((8, D), lambda i: (0, 0)),
        ],
        out_shape=[
            jax.ShapeDtypeStruct((8, D), jnp.float32),
            jax.ShapeDtypeStruct((8, D), jnp.float32),
        ],
    )(agg2, nb, W, b, Wcp, bcp)


def kernel(h, edge_index, W1, b1, W2, b2, Wc, bc):
    src = edge_index[0]
    dst = edge_index[1]
    pad = E_PAD - E
    lane = jnp.arange(pad, dtype=jnp.int32) % NDUMMY
    src_p = jnp.concatenate([src, lane])
    dst_p = jnp.concatenate([dst, N + lane])
    src2d = src_p.reshape(NW, EPW)
    dst3d = dst_p.reshape(NW, NCH, CHUNK)

    ones128 = jnp.ones((CHUNK, D), jnp.float32)
    zeros128 = jnp.zeros((ZR, D), jnp.float32)

    deg2 = _sc_degree(dst3d, ones128, zeros128)
    xs, nb = _tc_norm_scale(deg2, h)
    agg1 = _sc_spmm(xs, src2d, dst3d, zeros128)
    y1s = _tc_layer(agg1, nb, W1, b1.reshape(1, D))
    agg2 = _sc_spmm(y1s, src2d, dst3d, zeros128)

    wcp = jnp.zeros((D, D), jnp.float32).at[:, : Wc.shape[1]].set(Wc)
    bcp = jnp.zeros((1, D), jnp.float32).at[0, : bc.shape[0]].set(bc)
    _, logits = _tc_final(agg2, nb, W2, b2.reshape(1, D), wcp, bcp)
    return logits[0:1, : Wc.shape[1]]


# speedup vs baseline: 1.1730x; 1.0069x over previous
import functools

import jax
import jax.numpy as jnp
from jax import lax
from jax.experimental import pallas as pl
from jax.experimental.pallas import tpu as pltpu
from jax.experimental.pallas import tpu_sc as plsc

N = 10000
E = 320000
D = 128
NC = 2
NS = 16
NW = NC * NS
CHUNK = 96
NCH = 105
EPW = NCH * CHUNK
E_PAD = NW * EPW
CHUNK_D = 128
NCH_D = 79
EPW_D = NCH_D * CHUNK_D
E_PAD_D = NW * EPW_D
NDUMMY = 16
NPAD = 10240
RPT = NPAD // NS
LAST_RPT = N - 15 * RPT
ZR = 64
BR = 1000


def _sc_mesh():
    return plsc.VectorSubcoreMesh(core_axis_name="c", subcore_axis_name="s")


def _sc_degree(dst3d, ones128, zeros128):
    @functools.partial(
        pl.kernel,
        out_type=jax.ShapeDtypeStruct((NC, N, D), jnp.float32),
        mesh=_sc_mesh(),
        scratch_types=[
            pltpu.VMEM((NCH_D, CHUNK_D), jnp.int32),
            pltpu.VMEM((CHUNK_D, D), jnp.float32),
            pltpu.VMEM_SHARED((NPAD, D), jnp.float32),
        ],
    )
    def k(dst_hbm, ones_hbm, z_hbm, out_hbm, dst_v, ones_v, acc):
        cid = lax.axis_index("c")
        sid = lax.axis_index("s")
        wid = sid * NC + cid
        r0 = sid * RPT
        pltpu.sync_copy(z_hbm, ones_v.at[pl.ds(0, ZR)])
        for j in range(RPT // ZR):
            pltpu.sync_copy(ones_v.at[pl.ds(0, ZR)],
                            acc.at[pl.ds(r0 + j * ZR, ZR)])
        pltpu.sync_copy(ones_hbm, ones_v)
        pltpu.sync_copy(dst_hbm.at[wid], dst_v)
        plsc.subcore_barrier()

        def body(i, carry):
            pltpu.sync_copy(ones_v, acc.at[dst_v.at[i]], add=True)
            return carry

        lax.fori_loop(0, NCH_D, body, 0)
        plsc.subcore_barrier()

        @pl.when(sid < NS - 1)
        def _():
            pltpu.sync_copy(acc.at[pl.ds(r0, RPT)],
                            out_hbm.at[cid, pl.ds(r0, RPT)])

        @pl.when(sid == NS - 1)
        def _():
            pltpu.sync_copy(acc.at[pl.ds(r0, LAST_RPT)],
                            out_hbm.at[cid, pl.ds(r0, LAST_RPT)])

    return k(dst3d, ones128, zeros128)


def _sc_spmm(xs, src2d, dst3d, zeros128):
    @functools.partial(
        pl.kernel,
        out_type=jax.ShapeDtypeStruct((NC, N, D), jnp.float32),
        mesh=_sc_mesh(),
        scratch_types=[
            pltpu.VMEM((EPW,), jnp.int32),
            pltpu.VMEM((NCH, CHUNK), jnp.int32),
            pltpu.VMEM((CHUNK, D), jnp.float32),
            pltpu.VMEM((CHUNK, D), jnp.float32),
            pltpu.VMEM_SHARED((NPAD, D), jnp.float32),
            pltpu.SemaphoreType.DMA,
            pltpu.SemaphoreType.DMA,
        ],
    )
    def k(xs_hbm, src_hbm, dst_hbm, z_hbm, out_hbm,
          src_v, dst_v, rows0_v, rows1_v, acc, gsem0, gsem1):
        cid = lax.axis_index("c")
        sid = lax.axis_index("s")
        wid = sid * NC + cid
        r0 = sid * RPT
        pltpu.sync_copy(z_hbm, rows0_v.at[pl.ds(0, ZR)])
        for j in range(RPT // ZR):
            pltpu.sync_copy(rows0_v.at[pl.ds(0, ZR)],
                            acc.at[pl.ds(r0 + j * ZR, ZR)])
        pltpu.sync_copy(src_hbm.at[wid], src_v)
        pltpu.sync_copy(dst_hbm.at[wid], dst_v)
        plsc.subcore_barrier()

        bufs = (rows0_v, rows1_v)
        gsems = (gsem0, gsem1)

        def gather(i, b):
            pltpu.async_copy(
                xs_hbm.at[src_v.at[pl.ds(i * CHUNK, CHUNK)]], bufs[b], gsems[b]
            )

        def gwait(b):
            pltpu.make_async_copy(xs_hbm.at[pl.ds(0, CHUNK)], bufs[b],
                                  gsems[b]).wait()

        gather(0, 0)

        @pl.loop(0, NCH - 1, step=2)
        def _(g):
            for b in range(2):
                i = g + b
                gather(i + 1, 1 - b)
                gwait(b)
                pltpu.sync_copy(bufs[b], acc.at[dst_v.at[i]], add=True)

        gwait((NCH - 1) % 2)
        pltpu.sync_copy(bufs[(NCH - 1) % 2],
                        acc.at[dst_v.at[NCH - 1]], add=True)
        plsc.subcore_barrier()

        @pl.when(sid < NS - 1)
        def _():
            pltpu.sync_copy(acc.at[pl.ds(r0, RPT)],
                            out_hbm.at[cid, pl.ds(r0, RPT)])

        @pl.when(sid == NS - 1)
        def _():
            pltpu.sync_copy(acc.at[pl.ds(r0, LAST_RPT)],
                            out_hbm.at[cid, pl.ds(r0, LAST_RPT)])

    return k(xs, src2d, dst3d, zeros128)


def _tc_norm_scale(deg2, h):
    def body(deg_ref, h_ref, xs_ref, nb_ref):
        d = deg_ref[0, :, 0:1] + deg_ref[1, :, 0:1]
        norm = jnp.where(d > 0.0, lax.rsqrt(jnp.maximum(d, 1.0)), 0.0)
        nb = jnp.broadcast_to(norm, (BR, D))
        nb_ref[...] = nb
        xs_ref[...] = h_ref[...] * nb

    return pl.pallas_call(
        body,
        grid=(N // BR,),
        in_specs=[
            pl.BlockSpec((NC, BR, D), lambda i: (0, i, 0)),
            pl.BlockSpec((BR, D), lambda i: (i, 0)),
        ],
        out_specs=[
            pl.BlockSpec((BR, D), lambda i: (i, 0)),
            pl.BlockSpec((BR, D), lambda i: (i, 0)),
        ],
        out_shape=[
            jax.ShapeDtypeStruct((N, D), jnp.float32),
            jax.ShapeDtypeStruct((N, D), jnp.float32),
        ],
    )(deg2, h)


def _tc_layer(agg2, nb, W, b):
    def body(a_ref, nb_ref, w_ref, b_ref, o_ref):
        nbv = nb_ref[...]
        a = (a_ref[0] + a_ref[1]) * nbv
        z = jnp.dot(a, w_ref[...], preferred_element_type=jnp.float32)
        o_ref[...] = jnp.maximum(z + b_ref[...], 0.0) * nbv

    return pl.pallas_call(
        body,
        grid=(N // BR,),
        in_specs=[
            pl.BlockSpec((NC, BR, D), lambda i: (0, i, 0)),
            pl.BlockSpec((BR, D), lambda i: (i, 0)),
            pl.BlockSpec((D, D), lambda i: (0, 0)),
            pl.BlockSpec((1, D), lambda i: (0, 0)),
        ],
        out_specs=pl.BlockSpec((BR, D), lambda i: (i, 0)),
        out_shape=jax.ShapeDtypeStruct((N, D), jnp.float32),
    )(agg2, nb, W, b)


def _tc_final(agg2, nb, W, b, Wcp, bcp):
    def body(a_ref, nb_ref, w_ref, b_ref, wc_ref, bc_ref, pool_ref, o_ref):
        i = pl.program_id(0)
        a = (a_ref[0] + a_ref[1]) * nb_ref[...]
        z = jnp.dot(a, w_ref[...], preferred_element_type=jnp.float32)
        y = jnp.maximum(z + b_ref[...], 0.0)
        m = jnp.broadcast_to(jnp.max(y, axis=0, keepdims=True), (8, D))

        @pl.when(i == 0)
        def _():
            pool_ref[...] = m

        @pl.when(i > 0)
        def _():
            pool_ref[...] = jnp.maximum(pool_ref[...], m)

        @pl.when(i == pl.num_programs(0) - 1)
        def _():
            pooled = pool_ref[0:1, :]
            logits = jnp.dot(pooled, wc_ref[...],
                             preferred_element_type=jnp.float32) + bc_ref[...]
            o_ref[...] = jnp.broadcast_to(logits, (8, D))

    return pl.pallas_call(
        body,
        grid=(N // BR,),
        in_specs=[
            pl.BlockSpec((NC, BR, D), lambda i: (0, i, 0)),
            pl.BlockSpec((BR, D), lambda i: (i, 0)),
            pl.BlockSpec((D, D), lambda i: (0, 0)),
            pl.BlockSpec((1, D), lambda i: (0, 0)),
            pl.BlockSpec((D, D), lambda i: (0, 0)),
            pl.BlockSpec((1, D), lambda i: (0, 0)),
        ],
        out_specs=[
            pl.BlockSpec((8, D), lambda i: (0, 0)),
            pl.BlockSpec((8, D), lambda i: (0, 0)),
        ],
        out_shape=[
            jax.ShapeDtypeStruct((8, D), jnp.float32),
            jax.ShapeDtypeStruct((8, D), jnp.float32),
        ],
    )(agg2, nb, W, b, Wcp, bcp)


def kernel(h, edge_index, W1, b1, W2, b2, Wc, bc):
    src = edge_index[0]
    dst = edge_index[1]
    pad = E_PAD - E
    lane = jnp.arange(pad, dtype=jnp.int32) % NDUMMY
    src_p = jnp.concatenate([src, lane])
    dst_p = jnp.concatenate([dst, N + lane])
    src2d = src_p.reshape(NW, EPW)
    dst3d = dst_p.reshape(NW, NCH, CHUNK)

    pad_d = E_PAD_D - E
    lane_d = jnp.arange(pad_d, dtype=jnp.int32) % NDUMMY
    dst3d_d = jnp.concatenate([dst, N + lane_d]).reshape(NW, NCH_D, CHUNK_D)

    ones128 = jnp.ones((CHUNK_D, D), jnp.float32)
    zeros128 = jnp.zeros((ZR, D), jnp.float32)

    deg2 = _sc_degree(dst3d_d, ones128, zeros128)
    xs, nb = _tc_norm_scale(deg2, h)
    agg1 = _sc_spmm(xs, src2d, dst3d, zeros128)
    y1s = _tc_layer(agg1, nb, W1, b1.reshape(1, D))
    agg2 = _sc_spmm(y1s, src2d, dst3d, zeros128)

    wcp = jnp.zeros((D, D), jnp.float32).at[:, : Wc.shape[1]].set(Wc)
    bcp = jnp.zeros((1, D), jnp.float32).at[0, : bc.shape[0]].set(bc)
    _, logits = _tc_final(agg2, nb, W2, b2.reshape(1, D), wcp, bcp)
    return logits[0:1, : Wc.shape[1]]
